# baseline (speedup 1.0000x reference)
import functools

import jax
import jax.numpy as jnp
from jax import lax
from jax.experimental import pallas as pl
from jax.experimental.pallas import tpu as pltpu
from jax.experimental.pallas import tpu_sc as plsc

_SPATIAL_SCALE = 0.125
_TRANS_STD = 0.1
_B, _C, _H, _W = 2, 64, 256, 256
_N = 512
_P = 7
_NBINS = _P * _P
_NW = 32
_RPW = _N // _NW


def _round_half_even_scalar(x):
    k = x.astype(jnp.int32)
    kf = k.astype(jnp.float32)
    frac = x - kf
    up = (frac > 0.5).astype(jnp.int32)
    half_odd = jnp.where(frac == 0.5, k & 1, 0)
    return (k + up + half_odd).astype(jnp.float32)


def _deform_pool_sc(data_t, rois_pad, off_pad):
    mesh = plsc.VectorSubcoreMesh(core_axis_name="c", subcore_axis_name="s")

    @functools.partial(
        pl.kernel,
        mesh=mesh,
        compiler_params=pltpu.CompilerParams(use_tc_tiling_on_sc=False),
        out_type=jax.ShapeDtypeStruct((_N, _NBINS, _C), jnp.float32),
        scratch_types=[
            pltpu.VMEM((16,), jnp.float32),
            pltpu.VMEM((120,), jnp.float32),
            ]
            + [pltpu.VMEM((64,), jnp.int32) for _ in range(8)]
            + [pltpu.VMEM((64, _C), jnp.float32) for _ in range(8)]
            + [pltpu.VMEM((128,), jnp.float32) for _ in range(8)]
            + [pltpu.VMEM((_NBINS, _C), jnp.float32)]
            + [pltpu.SemaphoreType.DMA for _ in range(8)],
    )
    def kern(data_hbm, rois_hbm, off_hbm, out_hbm, roi_v, off_v,
             i0, i1, i2, i3, i4, i5, i6, i7,
             b0, b1, b2, b3, b4, b5, b6, b7,
             w0, w1, w2, w3, w4, w5, w6, w7,
             out_v, s0, s1, s2, s3, s4, s5, s6, s7):
        slots = list(zip([i0, i1, i2, i3, i4, i5, i6, i7],
                         [b0, b1, b2, b3, b4, b5, b6, b7],
                         [s0, s1, s2, s3, s4, s5, s6, s7],
                         [w0, w1, w2, w3, w4, w5, w6, w7]))
        wid = lax.axis_index("s") * 2 + lax.axis_index("c")
        r0 = wid * _RPW

        lane = lax.iota(jnp.int32, 16)
        iw_f = (lane & 3).astype(jnp.float32)
        ih_f = (lane >> 2).astype(jnp.float32)

        def roi_body(i, _):
            n = r0 + i
            pltpu.sync_copy(rois_hbm.at[n], roi_v)
            pltpu.sync_copy(off_hbm.at[n], off_v)

            rv = roi_v[...]
            base = rv[0].astype(jnp.int32) * (_H * _W)
            x1 = _round_half_even_scalar(rv[1])
            y1 = _round_half_even_scalar(rv[2])
            x2 = _round_half_even_scalar(rv[3])
            y2 = _round_half_even_scalar(rv[4])
            rsw = x1 * _SPATIAL_SCALE - 0.5
            rsh = y1 * _SPATIAL_SCALE - 0.5
            rew = (x2 + 1.0) * _SPATIAL_SCALE - 0.5
            reh = (y2 + 1.0) * _SPATIAL_SCALE - 0.5
            rw = jnp.maximum(rew - rsw, 0.1)
            rh = jnp.maximum(reh - rsh, 0.1)
            bw = rw * (1.0 / _P)
            bh = rh * (1.0 / _P)
            sw = bw * 0.25
            sh = bh * 0.25

            def issue_bin(b, idx_ref, rows_ref, sem, w_ref):
                ph = b // _P
                pw = b % _P
                tx = off_v[pl.ds(b, 16)][0] * _TRANS_STD
                ty = off_v[pl.ds(_NBINS + b, 16)][0] * _TRANS_STD
                wstart = pw.astype(jnp.float32) * bw + rsw + tx * rw
                hstart = ph.astype(jnp.float32) * bh + rsh + ty * rh

                w_vec = wstart + iw_f * sw
                h_vec = hstart + ih_f * sh
                mask = ((w_vec > -0.5) & (w_vec < _W - 0.5)
                        & (h_vec > -0.5) & (h_vec < _H - 0.5))
                wc = jnp.clip(w_vec, 0.0, _W - 1.0)
                hc = jnp.clip(h_vec, 0.0, _H - 1.0)
                w0 = wc.astype(jnp.int32)
                h0 = hc.astype(jnp.int32)
                lw = wc - w0.astype(jnp.float32)
                lh = hc - h0.astype(jnp.float32)
                hw = 1.0 - lw
                hh = 1.0 - lh
                w1 = jnp.minimum(w0 + 1, _W - 1)
                h1 = jnp.minimum(h0 + 1, _H - 1)
                row0 = base + h0 * _W
                row1 = base + h1 * _W
                idx_ref[pl.ds(0, 16)] = row0 + w0
                idx_ref[pl.ds(16, 16)] = row0 + w1
                idx_ref[pl.ds(32, 16)] = row1 + w0
                idx_ref[pl.ds(48, 16)] = row1 + w1
                pltpu.async_copy(data_hbm.at[idx_ref], rows_ref, sem)

                zero = jnp.zeros((16,), jnp.float32)
                maskf = jnp.where(mask, 1.0, zero)
                cnt = maskf[0]
                for k in range(1, 16):
                    cnt = cnt + maskf[k]
                cnt_vec = jnp.broadcast_to(jnp.maximum(cnt, 1.0), (16,))
                w_ref[pl.ds(0, 16)] = jnp.where(mask, hh * hw, zero)
                w_ref[pl.ds(16, 16)] = jnp.where(mask, hh * lw, zero)
                w_ref[pl.ds(32, 16)] = jnp.where(mask, lh * hw, zero)
                w_ref[pl.ds(48, 16)] = jnp.where(mask, lh * lw, zero)
                w_ref[pl.ds(64, 16)] = jnp.ones((16,), jnp.float32) / cnt_vec

            def combine_bin(b, w_ref, idx_ref, rows_ref, sem):
                pltpu.make_async_copy(
                    data_hbm.at[idx_ref], rows_ref, sem).wait()
                wgt = [w_ref[pl.ds(c4 * 16, 16)] for c4 in range(5)]
                zero = jnp.zeros((16,), jnp.float32)
                accs = [[zero] * 4 for _ in range(4)]
                for c4 in range(4):
                    for k in range(16):
                        w_bc = jnp.broadcast_to(wgt[c4][k], (16,))
                        for g in range(4):
                            accs[g][c4] = accs[g][c4] + w_bc * rows_ref[
                                c4 * 16 + k, pl.ds(g * 16, 16)]
                brow = jnp.minimum(b, _NBINS - 1)

                @pl.when(b < _NBINS)
                def _():
                    for g in range(4):
                        a = ((accs[g][0] + accs[g][1])
                             + (accs[g][2] + accs[g][3]))
                        out_v[brow, pl.ds(g * 16, 16)] = a * wgt[4]

            def issue_slot(b, s):
                idx_ref, rows_ref, sem, w_ref = slots[s]
                issue_bin(b, idx_ref, rows_ref, sem, w_ref)

            def combine_slot(b, s):
                idx_ref, rows_ref, sem, w_ref = slots[s]
                combine_bin(b, w_ref, idx_ref, rows_ref, sem)

            for s in range(7):
                issue_slot(jnp.int32(s), s)

            def bin_oct(i, _):
                v = i * 8
                for j in range(8):
                    issue_slot(v + j + 7, (j + 7) % 8)
                    combine_slot(v + j, j)
                return 0

            lax.fori_loop(0, 6, bin_oct, 0)
            issue_slot(jnp.int32(55), 7)
            for j in range(8):
                combine_slot(jnp.int32(48 + j), j)
            pltpu.sync_copy(out_v, out_hbm.at[n])
            return 0

        lax.fori_loop(0, _RPW, roi_body, 0)

    return kern(data_t, rois_pad, off_pad)


def kernel(data, rois, offset):
    data_t = jnp.transpose(data, (0, 2, 3, 1)).reshape(_B * _H * _W, _C)
    rois_pad = jnp.concatenate(
        [rois, jnp.zeros((_N, 11), jnp.float32)], axis=1)
    off_flat = offset.reshape(_N, 2 * _NBINS)
    off_pad = jnp.concatenate(
        [off_flat, jnp.zeros((_N, 120 - 2 * _NBINS), jnp.float32)], axis=1)
    out = _deform_pool_sc(data_t, rois_pad, off_pad)
    return out.reshape(_N, _P, _P, _C).transpose(0, 3, 1, 2)


# speedup vs baseline: 1.1295x; 1.1295x over previous
import functools

import jax
import jax.numpy as jnp
from jax import lax
from jax.experimental import pallas as pl
from jax.experimental.pallas import tpu as pltpu
from jax.experimental.pallas import tpu_sc as plsc

_SPATIAL_SCALE = 0.125
_TRANS_STD = 0.1
_B, _C, _H, _W = 2, 64, 256, 256
_N = 512
_P = 7
_NBINS = _P * _P
_NW = 32
_RPW = _N // _NW


def _round_half_even_scalar(x):
    k = x.astype(jnp.int32)
    kf = k.astype(jnp.float32)
    frac = x - kf
    up = (frac > 0.5).astype(jnp.int32)
    half_odd = jnp.where(frac == 0.5, k & 1, 0)
    return (k + up + half_odd).astype(jnp.float32)


def _deform_pool_sc(data_t, rois_pad, off_pad):
    mesh = plsc.VectorSubcoreMesh(core_axis_name="c", subcore_axis_name="s")

    @functools.partial(
        pl.kernel,
        mesh=mesh,
        compiler_params=pltpu.CompilerParams(use_tc_tiling_on_sc=False),
        out_type=jax.ShapeDtypeStruct((_N, _NBINS, _C), jnp.float32),
        scratch_types=[
            pltpu.VMEM((16,), jnp.float32),
            pltpu.VMEM((120,), jnp.float32),
            pltpu.VMEM((64,), jnp.int32),
            pltpu.VMEM((64,), jnp.int32),
            pltpu.VMEM((64,), jnp.int32),
            pltpu.VMEM((64,), jnp.int32),
            pltpu.VMEM((64, _C), jnp.float32),
            pltpu.VMEM((64, _C), jnp.float32),
            pltpu.VMEM((64, _C), jnp.float32),
            pltpu.VMEM((64, _C), jnp.float32),
            pltpu.VMEM((128,), jnp.float32),
            pltpu.VMEM((128,), jnp.float32),
            pltpu.VMEM((128,), jnp.float32),
            pltpu.VMEM((128,), jnp.float32),
            pltpu.VMEM((_NBINS, _C), jnp.float32),
            pltpu.SemaphoreType.DMA,
            pltpu.SemaphoreType.DMA,
            pltpu.SemaphoreType.DMA,
            pltpu.SemaphoreType.DMA,
        ],
    )
    def kern(data_hbm, rois_hbm, off_hbm, out_hbm,
             roi_v, off_v, idx0_v, idx1_v, idx2_v, idx3_v,
             rows0_v, rows1_v, rows2_v, rows3_v,
             w0_v, w1_v, w2_v, w3_v, out_v, sem0, sem1, sem2, sem3):
        slots = [(idx0_v, rows0_v, sem0, w0_v),
                 (idx1_v, rows1_v, sem1, w1_v),
                 (idx2_v, rows2_v, sem2, w2_v),
                 (idx3_v, rows3_v, sem3, w3_v)]
        wid = lax.axis_index("s") * 2 + lax.axis_index("c")
        r0 = wid * _RPW

        lane = lax.iota(jnp.int32, 16)
        iw_f = (lane & 3).astype(jnp.float32)
        ih_f = (lane >> 2).astype(jnp.float32)

        def roi_body(i, _):
            n = r0 + i
            pltpu.sync_copy(rois_hbm.at[n], roi_v)
            pltpu.sync_copy(off_hbm.at[n], off_v)

            rv = roi_v[...]
            base = rv[0].astype(jnp.int32) * (_H * _W)
            x1 = _round_half_even_scalar(rv[1])
            y1 = _round_half_even_scalar(rv[2])
            x2 = _round_half_even_scalar(rv[3])
            y2 = _round_half_even_scalar(rv[4])
            rsw = x1 * _SPATIAL_SCALE - 0.5
            rsh = y1 * _SPATIAL_SCALE - 0.5
            rew = (x2 + 1.0) * _SPATIAL_SCALE - 0.5
            reh = (y2 + 1.0) * _SPATIAL_SCALE - 0.5
            rw = jnp.maximum(rew - rsw, 0.1)
            rh = jnp.maximum(reh - rsh, 0.1)
            bw = rw * (1.0 / _P)
            bh = rh * (1.0 / _P)
            sw = bw * 0.25
            sh = bh * 0.25

            def issue_bin(b, idx_ref, rows_ref, sem, w_ref):
                ph = b // _P
                pw = b % _P
                tx = off_v[pl.ds(b, 16)][0] * _TRANS_STD
                ty = off_v[pl.ds(_NBINS + b, 16)][0] * _TRANS_STD
                wstart = pw.astype(jnp.float32) * bw + rsw + tx * rw
                hstart = ph.astype(jnp.float32) * bh + rsh + ty * rh

                w_vec = wstart + iw_f * sw
                h_vec = hstart + ih_f * sh
                mask_w = (w_vec > -0.5) & (w_vec < _W - 0.5)
                mask_h = (h_vec > -0.5) & (h_vec < _H - 0.5)
                mask = mask_w & mask_h
                wc = jnp.clip(w_vec, 0.0, _W - 1.0)
                hc = jnp.clip(h_vec, 0.0, _H - 1.0)
                w0 = wc.astype(jnp.int32)
                h0 = hc.astype(jnp.int32)
                lw = wc - w0.astype(jnp.float32)
                lh = hc - h0.astype(jnp.float32)
                hw = 1.0 - lw
                hh = 1.0 - lh
                w1 = jnp.minimum(w0 + 1, _W - 1)
                h1 = jnp.minimum(h0 + 1, _H - 1)
                row0 = base + h0 * _W
                row1 = base + h1 * _W
                idx_ref[pl.ds(0, 16)] = row0 + w0
                idx_ref[pl.ds(16, 16)] = row0 + w1
                idx_ref[pl.ds(32, 16)] = row1 + w0
                idx_ref[pl.ds(48, 16)] = row1 + w1
                pltpu.async_copy(data_hbm.at[idx_ref], rows_ref, sem)

                zero = jnp.zeros((16,), jnp.float32)
                mwf = jnp.where(mask_w, 1.0, zero)
                mhf = jnp.where(mask_h, 1.0, zero)
                cw = (mwf[0] + mwf[1]) + (mwf[2] + mwf[3])
                ch = (mhf[0] + mhf[4]) + (mhf[8] + mhf[12])
                cnt = cw * ch
                cnt_vec = jnp.broadcast_to(jnp.maximum(cnt, 1.0), (16,))
                w_ref[pl.ds(0, 16)] = jnp.where(mask, hh * hw, zero)
                w_ref[pl.ds(16, 16)] = jnp.where(mask, hh * lw, zero)
                w_ref[pl.ds(32, 16)] = jnp.where(mask, lh * hw, zero)
                w_ref[pl.ds(48, 16)] = jnp.where(mask, lh * lw, zero)
                w_ref[pl.ds(64, 16)] = jnp.ones((16,), jnp.float32) / cnt_vec

            def combine_bin(b, w_ref, idx_ref, rows_ref, sem):
                pltpu.make_async_copy(
                    data_hbm.at[idx_ref], rows_ref, sem).wait()
                wgt = [w_ref[pl.ds(c4 * 16, 16)] for c4 in range(5)]
                zero = jnp.zeros((16,), jnp.float32)
                accs = [[zero] * 4 for _ in range(4)]
                for c4 in range(4):
                    for k in range(16):
                        w_bc = jnp.broadcast_to(wgt[c4][k], (16,))
                        for g in range(4):
                            accs[g][c4] = accs[g][c4] + w_bc * rows_ref[
                                c4 * 16 + k, pl.ds(g * 16, 16)]
                brow = jnp.minimum(b, _NBINS - 1)

                @pl.when(b < _NBINS)
                def _():
                    for g in range(4):
                        a = ((accs[g][0] + accs[g][1])
                             + (accs[g][2] + accs[g][3]))
                        out_v[brow, pl.ds(g * 16, 16)] = a * wgt[4]

            def issue_slot(b, s):
                idx_ref, rows_ref, sem, w_ref = slots[s]
                issue_bin(b, idx_ref, rows_ref, sem, w_ref)

            def combine_slot(b, s):
                idx_ref, rows_ref, sem, w_ref = slots[s]
                combine_bin(b, w_ref, idx_ref, rows_ref, sem)

            for s in range(3):
                issue_slot(jnp.int32(s), s)

            def bin_quad(i, _):
                v = i * 4
                for j in range(4):
                    issue_slot(v + j + 3, (j + 3) % 4)
                    combine_slot(v + j, j)
                return 0

            lax.fori_loop(0, 12, bin_quad, 0)
            issue_slot(jnp.int32(51), 3)
            for j in range(4):
                combine_slot(jnp.int32(48 + j), j)
            pltpu.sync_copy(out_v, out_hbm.at[n])
            return 0

        lax.fori_loop(0, _RPW, roi_body, 0)

    return kern(data_t, rois_pad, off_pad)


def kernel(data, rois, offset):
    data_t = jnp.transpose(data, (0, 2, 3, 1)).reshape(_B * _H * _W, _C)
    rois_pad = jnp.concatenate(
        [rois, jnp.zeros((_N, 11), jnp.float32)], axis=1)
    off_flat = offset.reshape(_N, 2 * _NBINS)
    off_pad = jnp.concatenate(
        [off_flat, jnp.zeros((_N, 120 - 2 * _NBINS), jnp.float32)], axis=1)
    out = _deform_pool_sc(data_t, rois_pad, off_pad)
    return out.reshape(_N, _P, _P, _C).transpose(0, 3, 1, 2)
